# Initial kernel scaffold; baseline (speedup 1.0000x reference)
import jax
import jax.numpy as jnp
from jax.experimental import pallas as pl


def kernel(x, edge_index, edge_attr, lin_w, lin_b, root_w, bias):
    raise NotImplementedError("write your pallas kernel here")


# speedup vs baseline: 10.4777x; 10.4777x over previous
import functools

import jax
import jax.numpy as jnp
from jax import lax
from jax.experimental import pallas as pl
from jax.experimental.pallas import tpu as pltpu
from jax.experimental.pallas import tpu_sc as plsc

AD = 128
OD = 128
ED = 16
N = 10000
E = 320000

NC, NS = 2, 16
NW = NC * NS
EPW = E // NW
CH = 80
NCHUNK = EPW // CH
CZ = 16
NZC = N // CZ

_mesh = plsc.VectorSubcoreMesh(
    core_axis_name="c", subcore_axis_name="s", num_cores=NC, num_subcores=NS)


@functools.partial(
    pl.kernel,
    out_type=jax.ShapeDtypeStruct((E, AD), jnp.float32),
    mesh=_mesh,
    scratch_types=[
        pltpu.VMEM((CH,), jnp.int32),
        pltpu.VMEM((CH, AD), jnp.float32),
        pltpu.SemaphoreType.DMA,
    ],
)
def _gather_rows(x_hbm, src_hbm, xj_hbm, idx_v, rows_v, sem):
    wid = lax.axis_index("s") * NC + lax.axis_index("c")
    base = wid * EPW

    def body(j, carry):
        off = base + j * CH
        pltpu.sync_copy(src_hbm.at[pl.ds(off, CH)], idx_v)
        pltpu.async_copy(x_hbm.at[idx_v], rows_v, sem).wait()
        pltpu.sync_copy(rows_v, xj_hbm.at[pl.ds(off, CH)])
        return carry

    lax.fori_loop(0, NCHUNK, body, 0)


BE = 1280


def _msg_body(attr_ref, xj_ref, w_ref, out_ref):
    xj = xj_ref[...]
    attr = attr_ref[...]
    acc = jnp.dot(xj, w_ref[ED * AD:, :], preferred_element_type=jnp.float32)
    for k in range(ED):
        yk = jnp.dot(xj, w_ref[k * AD:(k + 1) * AD, :],
                     preferred_element_type=jnp.float32)
        acc = acc + attr[:, k][:, None] * yk
    out_ref[...] = acc


def _messages(attr, xj, wcat):
    return pl.pallas_call(
        _msg_body,
        grid=(E // BE,),
        in_specs=[
            pl.BlockSpec((BE, ED), lambda i: (i, 0)),
            pl.BlockSpec((BE, AD), lambda i: (i, 0)),
            pl.BlockSpec(((ED + 1) * AD, OD), lambda i: (0, 0)),
        ],
        out_specs=pl.BlockSpec((BE, OD), lambda i: (i, 0)),
        out_shape=jax.ShapeDtypeStruct((E, OD), jnp.float32),
    )(attr, xj, wcat)


@functools.partial(
    pl.kernel,
    out_type=jax.ShapeDtypeStruct((NC, N, OD), jnp.float32),
    mesh=_mesh,
    scratch_types=[
        pltpu.VMEM((CH,), jnp.int32),
        pltpu.VMEM((CH, OD), jnp.float32),
        pltpu.VMEM((CZ, OD), jnp.float32),
        pltpu.VMEM_SHARED((N, OD), jnp.float32),
        pltpu.SemaphoreType.DMA,
    ],
)
def _scatter_add(msg_hbm, dst_hbm, out_hbm, idx_v, rows_v, zero_v, acc_sh, sem):
    cid = lax.axis_index("c")
    sid = lax.axis_index("s")
    wid = sid * NC + cid
    base = wid * EPW

    zvec = jnp.zeros((16,), jnp.float32)

    def zfill(t, carry):
        zero_v[t // 8, pl.ds((t % 8) * 16, 16)] = zvec
        return carry

    lax.fori_loop(0, CZ * 8, zfill, 0)

    def zcopy(j, carry):
        c = j * NS + sid

        @pl.when(c < NZC)
        def _():
            pltpu.sync_copy(zero_v, acc_sh.at[pl.ds(c * CZ, CZ)])

        return carry

    lax.fori_loop(0, pl.cdiv(NZC, NS), zcopy, 0)
    plsc.subcore_barrier()

    def body(j, carry):
        off = base + j * CH
        pltpu.sync_copy(dst_hbm.at[pl.ds(off, CH)], idx_v)
        pltpu.sync_copy(msg_hbm.at[pl.ds(off, CH)], rows_v)
        pltpu.sync_copy(rows_v, acc_sh.at[idx_v], add=True)
        return carry

    lax.fori_loop(0, NCHUNK, body, 0)
    plsc.subcore_barrier()

    def dump(j, carry):
        c = j * NS + sid

        @pl.when(c < NZC)
        def _():
            pltpu.sync_copy(acc_sh.at[pl.ds(c * CZ, CZ)],
                            out_hbm.at[cid, pl.ds(c * CZ, CZ)])

        return carry

    lax.fori_loop(0, pl.cdiv(NZC, NS), dump, 0)


BN = 2000


def _final_body(p_ref, x_ref, rw_ref, b_ref, out_ref):
    root = jnp.dot(x_ref[...], rw_ref[...], preferred_element_type=jnp.float32)
    out_ref[...] = p_ref[0] + p_ref[1] + root + b_ref[...]


def _finalize(parts, x, root_w, bias2d):
    return pl.pallas_call(
        _final_body,
        grid=(N // BN,),
        in_specs=[
            pl.BlockSpec((NC, BN, OD), lambda i: (0, i, 0)),
            pl.BlockSpec((BN, AD), lambda i: (i, 0)),
            pl.BlockSpec((AD, OD), lambda i: (0, 0)),
            pl.BlockSpec((1, OD), lambda i: (0, 0)),
        ],
        out_specs=pl.BlockSpec((BN, OD), lambda i: (i, 0)),
        out_shape=jax.ShapeDtypeStruct((N, OD), jnp.float32),
    )(parts, x, root_w, bias2d)


def kernel(x, edge_index, edge_attr, lin_w, lin_b, root_w, bias):
    src = edge_index[0].astype(jnp.int32)
    dst = edge_index[1].astype(jnp.int32)
    wcat = jnp.concatenate(
        [lin_w.reshape(AD, OD, ED).transpose(2, 0, 1).reshape(ED * AD, OD),
         lin_b.reshape(AD, OD)], axis=0)
    xj = _gather_rows(x, src)
    msg = _messages(edge_attr, xj, wcat)
    parts = _scatter_add(msg, dst)
    return _finalize(parts, x, root_w, bias.reshape(1, OD))


# speedup vs baseline: 12.4378x; 1.1871x over previous
import functools

import jax
import jax.numpy as jnp
from jax import lax
from jax.experimental import pallas as pl
from jax.experimental.pallas import tpu as pltpu
from jax.experimental.pallas import tpu_sc as plsc

AD = 128
OD = 128
ED = 16
N = 10000
E = 320000

NC, NS = 2, 16
NW = NC * NS
EPW = E // NW
CH = 80
NCHUNK = EPW // CH
CZ = 16
NZC = N // CZ

_mesh = plsc.VectorSubcoreMesh(
    core_axis_name="c", subcore_axis_name="s", num_cores=NC, num_subcores=NS)


@functools.partial(
    pl.kernel,
    out_type=jax.ShapeDtypeStruct((E, AD), jnp.float32),
    mesh=_mesh,
    scratch_types=[
        pltpu.VMEM((CH,), jnp.int32),
        pltpu.VMEM((CH, AD), jnp.float32),
        pltpu.SemaphoreType.DMA,
    ],
)
def _gather_rows(x_hbm, src_hbm, xj_hbm, idx_v, rows_v, sem):
    wid = lax.axis_index("s") * NC + lax.axis_index("c")
    base = wid * EPW

    def body(j, carry):
        off = base + j * CH
        pltpu.sync_copy(src_hbm.at[pl.ds(off, CH)], idx_v)
        pltpu.async_copy(x_hbm.at[idx_v], rows_v, sem).wait()
        pltpu.sync_copy(rows_v, xj_hbm.at[pl.ds(off, CH)])
        return carry

    lax.fori_loop(0, NCHUNK, body, 0)


BE = 1600


def _msg_body(attr_ref, xj_ref, w_ref, out_ref):
    xj = xj_ref[...].astype(jnp.bfloat16)
    attr = attr_ref[...].astype(jnp.bfloat16)
    z = jnp.concatenate(
        [xj * attr[:, k][:, None] for k in range(ED)] + [xj], axis=1)
    out_ref[...] = jax.lax.dot_general(
        z, w_ref[...], (((1,), (0,)), ((), ())),
        preferred_element_type=jnp.float32)


def _messages(attr, xj, wcat):
    return pl.pallas_call(
        _msg_body,
        grid=(E // BE,),
        in_specs=[
            pl.BlockSpec((BE, ED), lambda i: (i, 0)),
            pl.BlockSpec((BE, AD), lambda i: (i, 0)),
            pl.BlockSpec(((ED + 1) * AD, OD), lambda i: (0, 0)),
        ],
        out_specs=pl.BlockSpec((BE, OD), lambda i: (i, 0)),
        out_shape=jax.ShapeDtypeStruct((E, OD), jnp.float32),
    )(attr, xj, wcat)


@functools.partial(
    pl.kernel,
    out_type=jax.ShapeDtypeStruct((NC, N, OD), jnp.float32),
    mesh=_mesh,
    scratch_types=[
        pltpu.VMEM((CH,), jnp.int32),
        pltpu.VMEM((CH, OD), jnp.float32),
        pltpu.VMEM((CZ, OD), jnp.float32),
        pltpu.VMEM_SHARED((N, OD), jnp.float32),
        pltpu.SemaphoreType.DMA,
    ],
)
def _scatter_add(msg_hbm, dst_hbm, out_hbm, idx_v, rows_v, zero_v, acc_sh, sem):
    cid = lax.axis_index("c")
    sid = lax.axis_index("s")
    wid = sid * NC + cid
    base = wid * EPW

    zvec = jnp.zeros((16,), jnp.float32)

    def zfill(t, carry):
        zero_v[t // 8, pl.ds((t % 8) * 16, 16)] = zvec
        return carry

    lax.fori_loop(0, CZ * 8, zfill, 0)

    def zcopy(j, carry):
        c = j * NS + sid

        @pl.when(c < NZC)
        def _():
            pltpu.sync_copy(zero_v, acc_sh.at[pl.ds(c * CZ, CZ)])

        return carry

    lax.fori_loop(0, pl.cdiv(NZC, NS), zcopy, 0)
    plsc.subcore_barrier()

    def body(j, carry):
        off = base + j * CH
        pltpu.sync_copy(dst_hbm.at[pl.ds(off, CH)], idx_v)
        pltpu.sync_copy(msg_hbm.at[pl.ds(off, CH)], rows_v)
        pltpu.sync_copy(rows_v, acc_sh.at[idx_v], add=True)
        return carry

    lax.fori_loop(0, NCHUNK, body, 0)
    plsc.subcore_barrier()

    def dump(j, carry):
        c = j * NS + sid

        @pl.when(c < NZC)
        def _():
            pltpu.sync_copy(acc_sh.at[pl.ds(c * CZ, CZ)],
                            out_hbm.at[cid, pl.ds(c * CZ, CZ)])

        return carry

    lax.fori_loop(0, pl.cdiv(NZC, NS), dump, 0)


BN = 2000


def _final_body(p_ref, x_ref, rw_ref, b_ref, out_ref):
    root = jnp.dot(x_ref[...], rw_ref[...], preferred_element_type=jnp.float32)
    out_ref[...] = p_ref[0] + p_ref[1] + root + b_ref[...]


def _finalize(parts, x, root_w, bias2d):
    return pl.pallas_call(
        _final_body,
        grid=(N // BN,),
        in_specs=[
            pl.BlockSpec((NC, BN, OD), lambda i: (0, i, 0)),
            pl.BlockSpec((BN, AD), lambda i: (i, 0)),
            pl.BlockSpec((AD, OD), lambda i: (0, 0)),
            pl.BlockSpec((1, OD), lambda i: (0, 0)),
        ],
        out_specs=pl.BlockSpec((BN, OD), lambda i: (i, 0)),
        out_shape=jax.ShapeDtypeStruct((N, OD), jnp.float32),
    )(parts, x, root_w, bias2d)


def kernel(x, edge_index, edge_attr, lin_w, lin_b, root_w, bias):
    src = edge_index[0].astype(jnp.int32)
    dst = edge_index[1].astype(jnp.int32)
    wcat = jnp.concatenate(
        [lin_w.reshape(AD, OD, ED).transpose(2, 0, 1).reshape(ED * AD, OD),
         lin_b.reshape(AD, OD)], axis=0).astype(jnp.bfloat16)
    xj = _gather_rows(x, src)
    msg = _messages(edge_attr, xj, wcat)
    parts = _scatter_add(msg, dst)
    return _finalize(parts, x, root_w, bias.reshape(1, OD))


# speedup vs baseline: 13.1582x; 1.0579x over previous
import functools

import jax
import jax.numpy as jnp
from jax import lax
from jax.experimental import pallas as pl
from jax.experimental.pallas import tpu as pltpu
from jax.experimental.pallas import tpu_sc as plsc

AD = 128
OD = 128
ED = 16
N = 10000
E = 320000

NH = 2
EH = E // NH

NC, NS = 2, 16
NW = NC * NS
EPW = EH // NW
CH = 40
NCHUNK = EPW // CH
CZ = 16
NZC = N // CZ

_mesh = plsc.VectorSubcoreMesh(
    core_axis_name="c", subcore_axis_name="s", num_cores=NC, num_subcores=NS)


@functools.partial(
    pl.kernel,
    out_type=jax.ShapeDtypeStruct((EH, AD), jnp.float32),
    mesh=_mesh,
    scratch_types=[
        pltpu.VMEM((CH,), jnp.int32),
        pltpu.VMEM((CH, AD), jnp.float32),
        pltpu.SemaphoreType.DMA,
    ],
)
def _gather_rows(x_hbm, src_hbm, xj_hbm, idx_v, rows_v, sem):
    wid = lax.axis_index("s") * NC + lax.axis_index("c")
    base = wid * EPW

    def body(j, carry):
        off = base + j * CH
        pltpu.sync_copy(src_hbm.at[pl.ds(off, CH)], idx_v)
        pltpu.async_copy(x_hbm.at[idx_v], rows_v, sem).wait()
        pltpu.sync_copy(rows_v, xj_hbm.at[pl.ds(off, CH)])
        return carry

    lax.fori_loop(0, NCHUNK, body, 0)


BE = 1600
KG = 8


def _msg_body(attr_ref, xj_ref, w_ref, out_ref):
    xj = xj_ref[...].astype(jnp.bfloat16)
    attr = attr_ref[...].astype(jnp.bfloat16)
    acc = jax.lax.dot_general(
        xj, w_ref[ED * AD:, :], (((1,), (0,)), ((), ())),
        preferred_element_type=jnp.float32)
    for g in range(ED // KG):
        zg = jnp.concatenate(
            [xj * attr[:, KG * g + j][:, None] for j in range(KG)], axis=1)
        acc = acc + jax.lax.dot_general(
            zg, w_ref[KG * g * AD:KG * (g + 1) * AD, :],
            (((1,), (0,)), ((), ())), preferred_element_type=jnp.float32)
    out_ref[...] = acc


def _messages(attr, xj, wcat):
    return pl.pallas_call(
        _msg_body,
        grid=(EH // BE,),
        in_specs=[
            pl.BlockSpec((BE, ED), lambda i: (i, 0)),
            pl.BlockSpec((BE, AD), lambda i: (i, 0)),
            pl.BlockSpec(((ED + 1) * AD, OD), lambda i: (0, 0)),
        ],
        out_specs=pl.BlockSpec((BE, OD), lambda i: (i, 0)),
        out_shape=jax.ShapeDtypeStruct((EH, OD), jnp.float32),
    )(attr, xj, wcat)


@functools.partial(
    pl.kernel,
    out_type=jax.ShapeDtypeStruct((NC, N, OD), jnp.float32),
    mesh=_mesh,
    scratch_types=[
        pltpu.VMEM((CH,), jnp.int32),
        pltpu.VMEM((CH, OD), jnp.float32),
        pltpu.VMEM((CZ, OD), jnp.float32),
        pltpu.VMEM_SHARED((N, OD), jnp.float32),
        pltpu.SemaphoreType.DMA,
    ],
)
def _scatter_add(msg_hbm, dst_hbm, out_hbm, idx_v, rows_v, zero_v, acc_sh, sem):
    cid = lax.axis_index("c")
    sid = lax.axis_index("s")
    wid = sid * NC + cid
    base = wid * EPW

    zvec = jnp.zeros((16,), jnp.float32)

    def zfill(t, carry):
        zero_v[t // 8, pl.ds((t % 8) * 16, 16)] = zvec
        return carry

    lax.fori_loop(0, CZ * 8, zfill, 0)

    def zcopy(j, carry):
        c = j * NS + sid

        @pl.when(c < NZC)
        def _():
            pltpu.sync_copy(zero_v, acc_sh.at[pl.ds(c * CZ, CZ)])

        return carry

    lax.fori_loop(0, pl.cdiv(NZC, NS), zcopy, 0)
    plsc.subcore_barrier()

    def body(j, carry):
        off = base + j * CH
        pltpu.sync_copy(dst_hbm.at[pl.ds(off, CH)], idx_v)
        pltpu.sync_copy(msg_hbm.at[pl.ds(off, CH)], rows_v)
        pltpu.sync_copy(rows_v, acc_sh.at[idx_v], add=True)
        return carry

    lax.fori_loop(0, NCHUNK, body, 0)
    plsc.subcore_barrier()

    def dump(j, carry):
        c = j * NS + sid

        @pl.when(c < NZC)
        def _():
            pltpu.sync_copy(acc_sh.at[pl.ds(c * CZ, CZ)],
                            out_hbm.at[cid, pl.ds(c * CZ, CZ)])

        return carry

    lax.fori_loop(0, pl.cdiv(NZC, NS), dump, 0)


BN = 2000
NP = NH * NC


def _final_body(p_ref, x_ref, rw_ref, b_ref, out_ref):
    root = jnp.dot(x_ref[...], rw_ref[...], preferred_element_type=jnp.float32)
    s = p_ref[0]
    for i in range(1, NP):
        s = s + p_ref[i]
    out_ref[...] = s + root + b_ref[...]


def _finalize(parts, x, root_w, bias2d):
    return pl.pallas_call(
        _final_body,
        grid=(N // BN,),
        in_specs=[
            pl.BlockSpec((NP, BN, OD), lambda i: (0, i, 0)),
            pl.BlockSpec((BN, AD), lambda i: (i, 0)),
            pl.BlockSpec((AD, OD), lambda i: (0, 0)),
            pl.BlockSpec((1, OD), lambda i: (0, 0)),
        ],
        out_specs=pl.BlockSpec((BN, OD), lambda i: (i, 0)),
        out_shape=jax.ShapeDtypeStruct((N, OD), jnp.float32),
    )(parts, x, root_w, bias2d)


def kernel(x, edge_index, edge_attr, lin_w, lin_b, root_w, bias):
    src = edge_index[0].astype(jnp.int32)
    dst = edge_index[1].astype(jnp.int32)
    wcat = jnp.concatenate(
        [lin_w.reshape(AD, OD, ED).transpose(2, 0, 1).reshape(ED * AD, OD),
         lin_b.reshape(AD, OD)], axis=0).astype(jnp.bfloat16)
    parts = []
    for h in range(NH):
        sl = slice(h * EH, (h + 1) * EH)
        xj_h = _gather_rows(x, src[sl])
        msg_h = _messages(edge_attr[sl], xj_h, wcat)
        parts.append(_scatter_add(msg_h, dst[sl]))
    parts_all = jnp.concatenate(parts, axis=0)
    return _finalize(parts_all, x, root_w, bias.reshape(1, OD))


# speedup vs baseline: 17.2507x; 1.3110x over previous
import functools

import jax
import jax.numpy as jnp
from jax import lax
from jax.experimental import pallas as pl
from jax.experimental.pallas import tpu as pltpu
from jax.experimental.pallas import tpu_sc as plsc

AD = 128
OD = 128
ED = 16
N = 10000
E = 320000

NH = 2
EH = E // NH

NC, NS = 2, 16
NW = NC * NS
EPW = EH // NW
CH = 40
NCHUNK = EPW // CH
NBUF = 5
CZ = 16
NZC = N // CZ

_mesh = plsc.VectorSubcoreMesh(
    core_axis_name="c", subcore_axis_name="s", num_cores=NC, num_subcores=NS)


def _make_gather(h):
    @functools.partial(
        pl.kernel,
        out_type=jax.ShapeDtypeStruct((EH, AD), jnp.float32),
        mesh=_mesh,
        scratch_types=[
            pltpu.VMEM((NBUF, CH), jnp.int32),
            pltpu.VMEM((NBUF, CH, AD), jnp.float32),
            pltpu.SemaphoreType.DMA,
            pltpu.SemaphoreType.DMA,
            pltpu.SemaphoreType.DMA,
        ],
    )
    def _gather_rows(x_hbm, src_hbm, xj_hbm, idx_v, rows_v, si, sg, sw):
        wid = lax.axis_index("s") * NC + lax.axis_index("c")
        base = wid * EPW

        def outer(t, carry):
            j0 = t * NBUF
            d = [pltpu.async_copy(
                src_hbm.at[pl.ds(h * EH + base + (j0 + b) * CH, CH)],
                idx_v.at[b], si) for b in range(NBUF)]
            for b in range(NBUF):
                d[b].wait()
            d = [pltpu.async_copy(x_hbm.at[idx_v.at[b]], rows_v.at[b], sg)
                 for b in range(NBUF)]
            for b in range(NBUF):
                d[b].wait()
            d = [pltpu.async_copy(
                rows_v.at[b], xj_hbm.at[pl.ds(base + (j0 + b) * CH, CH)], sw)
                for b in range(NBUF)]
            for b in range(NBUF):
                d[b].wait()
            return carry

        lax.fori_loop(0, NCHUNK // NBUF, outer, 0)

    return _gather_rows


_gathers = [_make_gather(h) for h in range(NH)]


BE = 1600
KG = 8


def _msg_body(attr_ref, xj_ref, w_ref, out_ref):
    xj = xj_ref[...].astype(jnp.bfloat16)
    attr = attr_ref[...].astype(jnp.bfloat16)
    acc = jax.lax.dot_general(
        xj, w_ref[ED * AD:, :], (((1,), (0,)), ((), ())),
        preferred_element_type=jnp.float32)
    for g in range(ED // KG):
        zg = jnp.concatenate(
            [xj * attr[:, KG * g + j][:, None] for j in range(KG)], axis=1)
        acc = acc + jax.lax.dot_general(
            zg, w_ref[KG * g * AD:KG * (g + 1) * AD, :],
            (((1,), (0,)), ((), ())), preferred_element_type=jnp.float32)
    out_ref[...] = acc


def _messages(attr, xj, wcat, h):
    nb = EH // BE
    return pl.pallas_call(
        _msg_body,
        grid=(nb,),
        in_specs=[
            pl.BlockSpec((BE, ED), lambda i: (i + h * nb, 0)),
            pl.BlockSpec((BE, AD), lambda i: (i, 0)),
            pl.BlockSpec(((ED + 1) * AD, OD), lambda i: (0, 0)),
        ],
        out_specs=pl.BlockSpec((BE, OD), lambda i: (i, 0)),
        out_shape=jax.ShapeDtypeStruct((EH, OD), jnp.float32),
    )(attr, xj, wcat)


def _make_scatter(h):
    @functools.partial(
        pl.kernel,
        out_type=jax.ShapeDtypeStruct((NC, N, OD), jnp.float32),
        mesh=_mesh,
        scratch_types=(
            [pltpu.VMEM((CH,), jnp.int32) for _ in range(NBUF)] + [
                pltpu.VMEM((NBUF, CH, OD), jnp.float32),
                pltpu.VMEM((CZ, OD), jnp.float32),
                pltpu.VMEM_SHARED((N, OD), jnp.float32),
                pltpu.SemaphoreType.DMA,
                pltpu.SemaphoreType.DMA,
                pltpu.SemaphoreType.DMA,
            ]),
    )
    def _scatter_add(msg_hbm, dst_hbm, out_hbm,
                     i0, i1, i2, i3, i4, rows_v, zero_v, acc_sh, si, sm, ss):
        idxs = [i0, i1, i2, i3, i4]
        cid = lax.axis_index("c")
        sid = lax.axis_index("s")
        wid = sid * NC + cid
        base = wid * EPW

        zvec = jnp.zeros((16,), jnp.float32)

        def zfill(t, carry):
            zero_v[t // 8, pl.ds((t % 8) * 16, 16)] = zvec
            return carry

        lax.fori_loop(0, CZ * 8, zfill, 0)

        def zcopy(j, carry):
            c = j * NS + sid

            @pl.when(c < NZC)
            def _():
                pltpu.sync_copy(zero_v, acc_sh.at[pl.ds(c * CZ, CZ)])

            return carry

        lax.fori_loop(0, pl.cdiv(NZC, NS), zcopy, 0)
        plsc.subcore_barrier()

        def outer(t, carry):
            j0 = t * NBUF
            di = [pltpu.async_copy(
                dst_hbm.at[pl.ds(h * EH + base + (j0 + b) * CH, CH)],
                idxs[b], si) for b in range(NBUF)]
            dm = [pltpu.async_copy(
                msg_hbm.at[pl.ds(base + (j0 + b) * CH, CH)],
                rows_v.at[b], sm) for b in range(NBUF)]
            for b in range(NBUF):
                di[b].wait()
                dm[b].wait()
            ds = [pltpu.async_copy(rows_v.at[b], acc_sh.at[idxs[b]], ss,
                                   add=True) for b in range(NBUF)]
            for b in range(NBUF):
                ds[b].wait()
            return carry

        lax.fori_loop(0, NCHUNK // NBUF, outer, 0)
        plsc.subcore_barrier()

        def dump(j, carry):
            c = j * NS + sid

            @pl.when(c < NZC)
            def _():
                pltpu.sync_copy(acc_sh.at[pl.ds(c * CZ, CZ)],
                                out_hbm.at[cid, pl.ds(c * CZ, CZ)])

            return carry

        lax.fori_loop(0, pl.cdiv(NZC, NS), dump, 0)

    return _scatter_add


_scatters = [_make_scatter(h) for h in range(NH)]


BN = 2000


def _final_body(p0_ref, p1_ref, x_ref, rw_ref, b_ref, out_ref):
    root = jnp.dot(x_ref[...], rw_ref[...], preferred_element_type=jnp.float32)
    s = p0_ref[0] + p0_ref[1] + p1_ref[0] + p1_ref[1]
    out_ref[...] = s + root + b_ref[...]


def _finalize(p0, p1, x, root_w, bias2d):
    return pl.pallas_call(
        _final_body,
        grid=(N // BN,),
        in_specs=[
            pl.BlockSpec((NC, BN, OD), lambda i: (0, i, 0)),
            pl.BlockSpec((NC, BN, OD), lambda i: (0, i, 0)),
            pl.BlockSpec((BN, AD), lambda i: (i, 0)),
            pl.BlockSpec((AD, OD), lambda i: (0, 0)),
            pl.BlockSpec((1, OD), lambda i: (0, 0)),
        ],
        out_specs=pl.BlockSpec((BN, OD), lambda i: (i, 0)),
        out_shape=jax.ShapeDtypeStruct((N, OD), jnp.float32),
    )(p0, p1, x, root_w, bias2d)


def kernel(x, edge_index, edge_attr, lin_w, lin_b, root_w, bias):
    src = edge_index[0].astype(jnp.int32)
    dst = edge_index[1].astype(jnp.int32)
    wcat = jnp.concatenate(
        [lin_w.reshape(AD, OD, ED).transpose(2, 0, 1).reshape(ED * AD, OD),
         lin_b.reshape(AD, OD)], axis=0).astype(jnp.bfloat16)
    parts = []
    for h in range(NH):
        xj_h = _gathers[h](x, src)
        msg_h = _messages(edge_attr, xj_h, wcat, h)
        parts.append(_scatters[h](msg_h, dst))
    return _finalize(parts[0], parts[1], x, root_w, bias.reshape(1, OD))


# speedup vs baseline: 20.4424x; 1.1850x over previous
import functools

import jax
import jax.numpy as jnp
from jax import lax
from jax.experimental import pallas as pl
from jax.experimental.pallas import tpu as pltpu
from jax.experimental.pallas import tpu_sc as plsc

AD = 128
OD = 128
ED = 16
N = 10000
E = 320000

NH = 2
EH = E // NH

NC, NS = 2, 16
NW = NC * NS
EPW = EH // NW
CH = 40
NCHUNK = EPW // CH
NBUF = 5
CZ = 16
NZC = N // CZ

_mesh = plsc.VectorSubcoreMesh(
    core_axis_name="c", subcore_axis_name="s", num_cores=NC, num_subcores=NS)


def _make_gather(h):
    @functools.partial(
        pl.kernel,
        out_type=jax.ShapeDtypeStruct((EH, AD), jnp.float32),
        mesh=_mesh,
        scratch_types=[
            pltpu.VMEM((NBUF, CH), jnp.int32),
            pltpu.VMEM((NBUF, CH, AD), jnp.float32),
            pltpu.SemaphoreType.DMA,
            pltpu.SemaphoreType.DMA,
            pltpu.SemaphoreType.DMA,
        ],
    )
    def _gather_rows(x_hbm, src_hbm, xj_hbm, idx_v, rows_v, si, sg, sw):
        wid = lax.axis_index("s") * NC + lax.axis_index("c")
        base = wid * EPW

        def outer(t, carry):
            j0 = t * NBUF
            d = [pltpu.async_copy(
                src_hbm.at[pl.ds(h * EH + base + (j0 + b) * CH, CH)],
                idx_v.at[b], si) for b in range(NBUF)]
            for b in range(NBUF):
                d[b].wait()
            d = [pltpu.async_copy(x_hbm.at[idx_v.at[b]], rows_v.at[b], sg)
                 for b in range(NBUF)]
            for b in range(NBUF):
                d[b].wait()
            d = [pltpu.async_copy(
                rows_v.at[b], xj_hbm.at[pl.ds(base + (j0 + b) * CH, CH)], sw)
                for b in range(NBUF)]
            for b in range(NBUF):
                d[b].wait()
            return carry

        lax.fori_loop(0, NCHUNK // NBUF, outer, 0)

    return _gather_rows


_gathers = [_make_gather(h) for h in range(NH)]


BE = 1280
KG = 8


def _msg_body(attrt_ref, xj_ref, w_ref, eye_ref, out_ref):
    xj = xj_ref[...].astype(jnp.bfloat16)
    xjt = jax.lax.dot_general(
        eye_ref[...], xj, (((1,), (1,)), ((), ())),
        preferred_element_type=jnp.float32).astype(jnp.bfloat16)
    attrt = attrt_ref[...].astype(jnp.bfloat16)
    acc = jax.lax.dot_general(
        xjt, w_ref[ED * AD:, :], (((0,), (0,)), ((), ())),
        preferred_element_type=jnp.float32)
    for g in range(ED // KG):
        zg = jnp.concatenate(
            [xjt * attrt[KG * g + j][None, :] for j in range(KG)], axis=0)
        acc = acc + jax.lax.dot_general(
            zg, w_ref[KG * g * AD:KG * (g + 1) * AD, :],
            (((0,), (0,)), ((), ())), preferred_element_type=jnp.float32)
    out_ref[...] = acc


def _messages(attrt, xj, wcat, eye, h):
    nb = EH // BE
    return pl.pallas_call(
        _msg_body,
        grid=(nb,),
        in_specs=[
            pl.BlockSpec((ED, BE), lambda i: (0, i + h * nb)),
            pl.BlockSpec((BE, AD), lambda i: (i, 0)),
            pl.BlockSpec(((ED + 1) * AD, OD), lambda i: (0, 0)),
            pl.BlockSpec((AD, AD), lambda i: (0, 0)),
        ],
        out_specs=pl.BlockSpec((BE, OD), lambda i: (i, 0)),
        out_shape=jax.ShapeDtypeStruct((EH, OD), jnp.float32),
    )(attrt, xj, wcat, eye)


def _make_scatter(h):
    @functools.partial(
        pl.kernel,
        out_type=jax.ShapeDtypeStruct((NC, N, OD), jnp.float32),
        mesh=_mesh,
        scratch_types=(
            [pltpu.VMEM((CH,), jnp.int32) for _ in range(NBUF)] + [
                pltpu.VMEM((NBUF, CH, OD), jnp.float32),
                pltpu.VMEM((CZ, OD), jnp.float32),
                pltpu.VMEM_SHARED((N, OD), jnp.float32),
                pltpu.SemaphoreType.DMA,
                pltpu.SemaphoreType.DMA,
                pltpu.SemaphoreType.DMA,
            ]),
    )
    def _scatter_add(msg_hbm, dst_hbm, out_hbm,
                     i0, i1, i2, i3, i4, rows_v, zero_v, acc_sh, si, sm, ss):
        idxs = [i0, i1, i2, i3, i4]
        cid = lax.axis_index("c")
        sid = lax.axis_index("s")
        wid = sid * NC + cid
        base = wid * EPW

        zvec = jnp.zeros((16,), jnp.float32)

        def zfill(t, carry):
            zero_v[t // 8, pl.ds((t % 8) * 16, 16)] = zvec
            return carry

        lax.fori_loop(0, CZ * 8, zfill, 0)

        def zcopy(j, carry):
            c = j * NS + sid

            @pl.when(c < NZC)
            def _():
                pltpu.sync_copy(zero_v, acc_sh.at[pl.ds(c * CZ, CZ)])

            return carry

        lax.fori_loop(0, pl.cdiv(NZC, NS), zcopy, 0)
        plsc.subcore_barrier()

        def outer(t, carry):
            j0 = t * NBUF
            di = [pltpu.async_copy(
                dst_hbm.at[pl.ds(h * EH + base + (j0 + b) * CH, CH)],
                idxs[b], si) for b in range(NBUF)]
            dm = [pltpu.async_copy(
                msg_hbm.at[pl.ds(base + (j0 + b) * CH, CH)],
                rows_v.at[b], sm) for b in range(NBUF)]
            for b in range(NBUF):
                di[b].wait()
                dm[b].wait()
            ds = [pltpu.async_copy(rows_v.at[b], acc_sh.at[idxs[b]], ss,
                                   add=True) for b in range(NBUF)]
            for b in range(NBUF):
                ds[b].wait()
            return carry

        lax.fori_loop(0, NCHUNK // NBUF, outer, 0)
        plsc.subcore_barrier()

        def dump(j, carry):
            c = j * NS + sid

            @pl.when(c < NZC)
            def _():
                pltpu.sync_copy(acc_sh.at[pl.ds(c * CZ, CZ)],
                                out_hbm.at[cid, pl.ds(c * CZ, CZ)])

            return carry

        lax.fori_loop(0, pl.cdiv(NZC, NS), dump, 0)

    return _scatter_add


_scatters = [_make_scatter(h) for h in range(NH)]


BN = 2000


def _final_body(p0_ref, p1_ref, x_ref, rw_ref, b_ref, out_ref):
    root = jnp.dot(x_ref[...], rw_ref[...], preferred_element_type=jnp.float32)
    s = p0_ref[0] + p0_ref[1] + p1_ref[0] + p1_ref[1]
    out_ref[...] = s + root + b_ref[...]


def _finalize(p0, p1, x, root_w, bias2d):
    return pl.pallas_call(
        _final_body,
        grid=(N // BN,),
        in_specs=[
            pl.BlockSpec((NC, BN, OD), lambda i: (0, i, 0)),
            pl.BlockSpec((NC, BN, OD), lambda i: (0, i, 0)),
            pl.BlockSpec((BN, AD), lambda i: (i, 0)),
            pl.BlockSpec((AD, OD), lambda i: (0, 0)),
            pl.BlockSpec((1, OD), lambda i: (0, 0)),
        ],
        out_specs=pl.BlockSpec((BN, OD), lambda i: (i, 0)),
        out_shape=jax.ShapeDtypeStruct((N, OD), jnp.float32),
    )(p0, p1, x, root_w, bias2d)


def kernel(x, edge_index, edge_attr, lin_w, lin_b, root_w, bias):
    src = edge_index[0].astype(jnp.int32)
    dst = edge_index[1].astype(jnp.int32)
    wcat = jnp.concatenate(
        [lin_w.reshape(AD, OD, ED).transpose(2, 0, 1).reshape(ED * AD, OD),
         lin_b.reshape(AD, OD)], axis=0).astype(jnp.bfloat16)
    attrt = edge_attr.T
    eye = jnp.eye(AD, dtype=jnp.bfloat16)
    parts = []
    for h in range(NH):
        xj_h = _gathers[h](x, src)
        msg_h = _messages(attrt, xj_h, wcat, eye, h)
        parts.append(_scatters[h](msg_h, dst))
    return _finalize(parts[0], parts[1], x, root_w, bias.reshape(1, OD))
